# baseline (speedup 1.0000x reference)
import functools

import jax
import jax.numpy as jnp
from jax import lax
from jax.experimental import pallas as pl
from jax.experimental.pallas import tpu as pltpu
from jax.experimental.pallas import tpu_sc as plsc

N = 10000
E = 160000
D = 256
DH = 128
NPAD = 10240
ROWS_PER_TILE = NPAD // 16
CHUNK = 64
BI = 16
NBLK = 10
NCH = BI * NBLK
E_TILE = NCH * CHUNK
EPAD = 16 * E_TILE
DUMMY = N

_MESH = plsc.VectorSubcoreMesh(core_axis_name="c", subcore_axis_name="s",
                               num_cores=2, num_subcores=16)


def _degree_body(sdidx_hbm, out_hbm, idx_v, ones_v, row_v, hist_s):
    c = lax.axis_index("c")
    s = lax.axis_index("s")

    def _fill_ones(i, carry):
        ones_v[pl.ds(i * 16, 16)] = jnp.ones((16,), jnp.float32)
        return carry

    lax.fori_loop(0, CHUNK // 16, _fill_ones, 0)

    def _fill_zero(i, carry):
        row_v[pl.ds(i * 16, 16)] = jnp.zeros((16,), jnp.float32)
        return carry

    lax.fori_loop(0, ROWS_PER_TILE // 16, _fill_zero, 0)
    pltpu.sync_copy(row_v, hist_s.at[pl.ds(s * ROWS_PER_TILE, ROWS_PER_TILE)])
    pltpu.sync_copy(sdidx_hbm.at[0, s], idx_v)
    plsc.subcore_barrier()

    def _accum(b, carry):
        for i in range(BI):
            pltpu.sync_copy(ones_v, hist_s.at[idx_v.at[b, i, c]], add=True)
        return carry

    lax.fori_loop(0, NBLK, _accum, 0)
    plsc.subcore_barrier()
    pltpu.sync_copy(hist_s.at[pl.ds(s * ROWS_PER_TILE, ROWS_PER_TILE)], row_v)
    pltpu.sync_copy(row_v, out_hbm.at[c, pl.ds(s * ROWS_PER_TILE, ROWS_PER_TILE)])


_degree_kernel = functools.partial(
    pl.kernel,
    out_type=jax.ShapeDtypeStruct((2, NPAD), jnp.float32),
    mesh=_MESH,
    scratch_types=[
        pltpu.VMEM((NBLK, BI, 2, CHUNK), jnp.int32),
        pltpu.VMEM((CHUNK,), jnp.float32),
        pltpu.VMEM((ROWS_PER_TILE,), jnp.float32),
        pltpu.VMEM_SHARED((NPAD,), jnp.float32),
    ],
)(_degree_body)


def _agg_body(table_hbm, sdidx_hbm, out_hbm,
              sd_v0, sd_v1, buf_a, buf_b, agg_s,
              sem_g0, sem_g1, sem_s0, sem_s1, sem_i0, sem_i1):
    c = lax.axis_index("c")
    s = lax.axis_index("s")
    bufs = (buf_a, buf_b)
    gsems = (sem_g0, sem_g1)
    ssems = (sem_s0, sem_s1)
    sds = (sd_v0, sd_v1)
    isems = (sem_i0, sem_i1)

    def _zero_row(i, carry):
        for l in range(DH // 16):
            buf_a[i, pl.ds(l * 16, 16)] = jnp.zeros((16,), jnp.float32)
        return carry

    lax.fori_loop(0, CHUNK, _zero_row, 0)
    for k in range(ROWS_PER_TILE // CHUNK):
        pltpu.sync_copy(buf_a, agg_s.at[pl.ds(s * ROWS_PER_TILE + k * CHUNK, CHUNK)])
    plsc.subcore_barrier()

    pltpu.sync_copy(sdidx_hbm.at[c, s, 0], sd_v0)
    pltpu.async_copy(sdidx_hbm.at[c, s, 1], sd_v1, sem_i1)
    pltpu.async_copy(table_hbm.at[sd_v0.at[0, 0]], buf_a, sem_g0)

    def _gather(sd, i, p):
        pltpu.async_copy(table_hbm.at[sd.at[i, 0]], bufs[p], gsems[p])

    def _wait_gather(sd, i, p):
        pltpu.make_async_copy(table_hbm.at[sd.at[i, 0]], bufs[p], gsems[p]).wait()

    def _wait_scatter(sd, i, p):
        pltpu.make_async_copy(bufs[p], agg_s.at[sd.at[i, 1]], ssems[p]).wait()

    def _dblk(t, carry):
        for half in range(2):
            sd = sds[half]
            nxt = sds[(half + 1) % 2]
            blk = 2 * t + half
            for i in range(BI):
                p = i % 2
                q = (i + 1) % 2
                _wait_gather(sd, i, p)
                pltpu.async_copy(bufs[p], agg_s.at[sd.at[i, 1]], ssems[p],
                                 add=True)
                if i == 0:
                    def _drain_prev():
                        _wait_scatter(sd, i, q)

                    def _prefetch_next():
                        pltpu.async_copy(sdidx_hbm.at[c, s, blk + 1], nxt,
                                         isems[(half + 1) % 2])

                    if half == 0:
                        pl.when(t > 0)(_drain_prev)
                        pl.when((t > 0) & (t < NBLK // 2))(_prefetch_next)
                    else:
                        _drain_prev()
                        pl.when(t < NBLK // 2 - 1)(_prefetch_next)
                else:
                    _wait_scatter(sd, i, q)
                if i < BI - 1:
                    _gather(sd, i + 1, q)
                else:
                    def _issue_next():
                        pltpu.make_async_copy(
                            sdidx_hbm.at[c, s, blk + 1], nxt,
                            isems[(half + 1) % 2]).wait()
                        _gather(nxt, 0, 0)

                    if half == 0:
                        _issue_next()
                    else:
                        pl.when(t < NBLK // 2 - 1)(_issue_next)
        return carry

    lax.fori_loop(0, NBLK // 2, _dblk, 0)
    _wait_scatter(sd_v1, BI - 1, 1)

    plsc.subcore_barrier()
    for k in range(ROWS_PER_TILE // CHUNK):
        off = s * ROWS_PER_TILE + k * CHUNK
        pltpu.sync_copy(agg_s.at[pl.ds(off, CHUNK)], buf_a)
        pltpu.sync_copy(buf_a, out_hbm.at[pl.ds(c * NPAD + off, CHUNK)])


_agg_kernel = functools.partial(
    pl.kernel,
    out_type=jax.ShapeDtypeStruct((2 * NPAD, DH), jnp.float32),
    mesh=_MESH,
    scratch_types=[
        pltpu.VMEM((BI, 2, CHUNK), jnp.int32),
        pltpu.VMEM((BI, 2, CHUNK), jnp.int32),
        pltpu.VMEM((CHUNK, DH), jnp.float32),
        pltpu.VMEM((CHUNK, DH), jnp.float32),
        pltpu.VMEM_SHARED((NPAD, DH), jnp.float32),
        pltpu.SemaphoreType.DMA,
        pltpu.SemaphoreType.DMA,
        pltpu.SemaphoreType.DMA,
        pltpu.SemaphoreType.DMA,
        pltpu.SemaphoreType.DMA,
        pltpu.SemaphoreType.DMA,
    ],
)(_agg_body)


_BN = 640


def _tc1_body(f_ref, w_ref, d_ref, o_ref):
    norm = lax.rsqrt(jnp.maximum(d_ref[...], 1.0))
    o_ref[...] = jnp.dot(f_ref[...], w_ref[...],
                         preferred_element_type=jnp.float32) * norm


def _tc2_body(a0_ref, a1_ref, w_ref, b_ref, dd_ref, ds_ref, o_ref):
    nd = lax.rsqrt(jnp.maximum(dd_ref[...], 1.0))
    ns = lax.rsqrt(jnp.maximum(ds_ref[...], 1.0))
    h = jnp.concatenate([a0_ref[...], a1_ref[...]], axis=1) * nd + b_ref[...]
    h = jnp.maximum(h, 0.0)
    o_ref[...] = jnp.dot(h, w_ref[...], preferred_element_type=jnp.float32) * ns


def _tc3_body(a0_ref, a1_ref, b_ref, dd_ref, o_ref):
    nd = lax.rsqrt(jnp.maximum(dd_ref[...], 1.0))
    h = jnp.concatenate([a0_ref[...], a1_ref[...]], axis=1) * nd + b_ref[...]
    o_ref[...] = jnp.maximum(h, 0.0)


def _tc1(feat, w, deg_src):
    return pl.pallas_call(
        _tc1_body,
        grid=(NPAD // _BN, 2),
        in_specs=[
            pl.BlockSpec((_BN, D), lambda i, c: (i, 0)),
            pl.BlockSpec((D, DH), lambda i, c: (0, c)),
            pl.BlockSpec((_BN, 1), lambda i, c: (i, 0)),
        ],
        out_specs=pl.BlockSpec((_BN, DH), lambda i, c: (c * (NPAD // _BN) + i, 0)),
        out_shape=jax.ShapeDtypeStruct((2 * NPAD, DH), jnp.float32),
    )(feat, w, deg_src)


def _tc2(agg, w, b, deg_dst, deg_src):
    nb = NPAD // _BN
    return pl.pallas_call(
        _tc2_body,
        grid=(nb, 2),
        in_specs=[
            pl.BlockSpec((_BN, DH), lambda i, c: (i, 0)),
            pl.BlockSpec((_BN, DH), lambda i, c: (nb + i, 0)),
            pl.BlockSpec((D, DH), lambda i, c: (0, c)),
            pl.BlockSpec((1, D), lambda i, c: (0, 0)),
            pl.BlockSpec((_BN, 1), lambda i, c: (i, 0)),
            pl.BlockSpec((_BN, 1), lambda i, c: (i, 0)),
        ],
        out_specs=pl.BlockSpec((_BN, DH), lambda i, c: (c * nb + i, 0)),
        out_shape=jax.ShapeDtypeStruct((2 * NPAD, DH), jnp.float32),
    )(agg, agg, w, b, deg_dst, deg_src)


def _tc3(agg, b, deg_dst):
    nb = NPAD // _BN
    return pl.pallas_call(
        _tc3_body,
        grid=(nb,),
        in_specs=[
            pl.BlockSpec((_BN, DH), lambda i: (i, 0)),
            pl.BlockSpec((_BN, DH), lambda i: (nb + i, 0)),
            pl.BlockSpec((1, D), lambda i: (0, 0)),
            pl.BlockSpec((_BN, 1), lambda i: (i, 0)),
        ],
        out_specs=pl.BlockSpec((_BN, D), lambda i: (i, 0)),
        out_shape=jax.ShapeDtypeStruct((N, D), jnp.float32),
    )(agg, agg, b, deg_dst)


def kernel(feat, edge_index, W1, b1, W2, b2):
    src = edge_index[0]
    dst = edge_index[1]
    pad = EPAD - E
    src_p = jnp.concatenate([src, jnp.full((pad,), DUMMY, jnp.int32)])
    dst_p = jnp.concatenate([dst, jnp.full((pad,), DUMMY, jnp.int32)])
    src4 = src_p.reshape(16, NBLK, BI, CHUNK)
    dst4 = dst_p.reshape(16, NBLK, BI, CHUNK)
    sdidx = jnp.stack([
        jnp.stack([src4, dst4], axis=3),
        jnp.stack([src4 + NPAD, dst4], axis=3),
    ])

    degs = _degree_kernel(sdidx)
    deg_src = degs[0].reshape(NPAD, 1)
    deg_dst = degs[1].reshape(NPAD, 1)

    b1r = b1.reshape(1, D)
    b2r = b2.reshape(1, D)

    hs1 = _tc1(feat, W1, deg_src)
    agg1 = _agg_kernel(hs1, sdidx)
    hs2 = _tc2(agg1, W2, b1r, deg_dst, deg_src)
    agg2 = _agg_kernel(hs2, sdidx)
    return _tc3(agg2, b2r, deg_dst)


# speedup vs baseline: 1.2460x; 1.2460x over previous
import functools

import jax
import jax.numpy as jnp
import numpy as np
from jax import lax
from jax.experimental import pallas as pl
from jax.experimental.pallas import tpu as pltpu
from jax.experimental.pallas import tpu_sc as plsc

N = 10000
E = 160000
D = 256
DH = 128
NPAD = 10240
ROWS_PER_TILE = NPAD // 16
CHUNK = 80
BI = 16
NBLK = 8
NBUF = 2
PIPE = NBUF - 1
NCH = BI * NBLK
E_TILE = NCH * CHUNK
EPAD = 16 * E_TILE
DUMMY = N

_MESH = plsc.VectorSubcoreMesh(core_axis_name="c", subcore_axis_name="s",
                               num_cores=2, num_subcores=16)


def _degree_body(sdidx_hbm, out_hbm, idx_v, ones_v, row_v, hist_s):
    c = lax.axis_index("c")
    s = lax.axis_index("s")

    def _fill_ones(i, carry):
        ones_v[pl.ds(i * 16, 16)] = jnp.ones((16,), jnp.float32)
        return carry

    lax.fori_loop(0, CHUNK // 16, _fill_ones, 0)

    def _fill_zero(i, carry):
        row_v[pl.ds(i * 16, 16)] = jnp.zeros((16,), jnp.float32)
        return carry

    lax.fori_loop(0, ROWS_PER_TILE // 16, _fill_zero, 0)
    pltpu.sync_copy(row_v, hist_s.at[pl.ds(s * ROWS_PER_TILE, ROWS_PER_TILE)])
    pltpu.sync_copy(sdidx_hbm.at[0, s], idx_v)
    plsc.subcore_barrier()

    def _accum(b, carry):
        for i in range(BI):
            pltpu.sync_copy(ones_v, hist_s.at[idx_v.at[b, i, c]], add=True)
        return carry

    lax.fori_loop(0, NBLK, _accum, 0)
    plsc.subcore_barrier()
    pltpu.sync_copy(hist_s.at[pl.ds(s * ROWS_PER_TILE, ROWS_PER_TILE)], row_v)
    pltpu.sync_copy(row_v, out_hbm.at[c, pl.ds(s * ROWS_PER_TILE, ROWS_PER_TILE)])


_degree_kernel = functools.partial(
    pl.kernel,
    out_type=jax.ShapeDtypeStruct((2, NPAD), jnp.float32),
    mesh=_MESH,
    scratch_types=[
        pltpu.VMEM((NBLK, BI, 2, CHUNK), jnp.int32),
        pltpu.VMEM((CHUNK,), jnp.float32),
        pltpu.VMEM((ROWS_PER_TILE,), jnp.float32),
        pltpu.VMEM_SHARED((NPAD,), jnp.float32),
    ],
)(_degree_body)


def _agg_body(table_hbm, sdidx_hbm, out_hbm,
              sd_v0, sd_v1, buf_a, buf_b, agg_s,
              sem_g0, sem_g1, sem_i0, sem_i1):
    c = lax.axis_index("c")
    s = lax.axis_index("s")
    bufs = (buf_a, buf_b)
    gsems = (sem_g0, sem_g1)
    sds = (sd_v0, sd_v1)
    isems = (sem_i0, sem_i1)

    def _zero_row(i, carry):
        for l in range(DH // 16):
            buf_a[i, pl.ds(l * 16, 16)] = jnp.zeros((16,), jnp.float32)
        return carry

    lax.fori_loop(0, CHUNK, _zero_row, 0)
    for k in range(ROWS_PER_TILE // CHUNK):
        pltpu.sync_copy(buf_a, agg_s.at[pl.ds(s * ROWS_PER_TILE + k * CHUNK, CHUNK)])
    plsc.subcore_barrier()

    def _gather(sd, i, p):
        pltpu.async_copy(table_hbm.at[sd.at[i, 0]], bufs[p], gsems[p])

    def _wait_gather(sd, i, p):
        pltpu.make_async_copy(table_hbm.at[sd.at[i, 0]], bufs[p], gsems[p]).wait()

    pltpu.sync_copy(sdidx_hbm.at[c, s, 0], sd_v0)
    pltpu.async_copy(sdidx_hbm.at[c, s, 1], sd_v1, sem_i1)
    for i in range(PIPE):
        _gather(sd_v0, i, i)

    def _dblk(t, carry):
        for half in range(2):
            sd = sds[half]
            nxt = sds[(half + 1) % 2]
            blk = 2 * t + half
            last = half == 1
            for i in range(BI):
                p = i % NBUF
                if i < BI - PIPE:
                    _gather(sd, i + PIPE, (i + PIPE) % NBUF)
                else:
                    ii = i + PIPE - BI

                    def _issue_next(ii=ii):
                        if ii == 0:
                            pltpu.make_async_copy(
                                sdidx_hbm.at[c, s, blk + 1], nxt,
                                isems[(half + 1) % 2]).wait()
                        _gather(nxt, ii, ii % NBUF)

                    if not last:
                        _issue_next()
                    else:
                        pl.when(t < NBLK // 2 - 1)(_issue_next)
                _wait_gather(sd, i, p)
                pltpu.sync_copy(bufs[p], agg_s.at[sd.at[i, 1]], add=True)

            def _prefetch():
                pltpu.async_copy(sdidx_hbm.at[c, s, blk + 2], sd,
                                 isems[half])

            pl.when(blk + 2 < NBLK)(_prefetch)
        return carry

    lax.fori_loop(0, NBLK // 2, _dblk, 0)

    plsc.subcore_barrier()
    for k in range(ROWS_PER_TILE // CHUNK):
        off = s * ROWS_PER_TILE + k * CHUNK
        pltpu.sync_copy(agg_s.at[pl.ds(off, CHUNK)], buf_a)
        pltpu.sync_copy(buf_a, out_hbm.at[pl.ds(c * NPAD + off, CHUNK)])


_agg_kernel = functools.partial(
    pl.kernel,
    out_type=jax.ShapeDtypeStruct((2 * NPAD, DH), jnp.float32),
    mesh=_MESH,
    scratch_types=[
        pltpu.VMEM((BI, 2, CHUNK), jnp.int32),
        pltpu.VMEM((BI, 2, CHUNK), jnp.int32),
        pltpu.VMEM((CHUNK, DH), jnp.float32),
        pltpu.VMEM((CHUNK, DH), jnp.float32),
        pltpu.VMEM_SHARED((NPAD, DH), jnp.float32),
        pltpu.SemaphoreType.DMA,
        pltpu.SemaphoreType.DMA,
        pltpu.SemaphoreType.DMA,
        pltpu.SemaphoreType.DMA,
    ],
)(_agg_body)


_BN = 640


def _tc1_body(f_ref, w_ref, d_ref, o_ref):
    norm = lax.rsqrt(jnp.maximum(d_ref[...], 1.0))
    o_ref[...] = jnp.dot(f_ref[...], w_ref[...],
                         preferred_element_type=jnp.float32) * norm


def _tc2_body(a0_ref, a1_ref, w_ref, b_ref, dd_ref, ds_ref, o_ref):
    nd = lax.rsqrt(jnp.maximum(dd_ref[...], 1.0))
    ns = lax.rsqrt(jnp.maximum(ds_ref[...], 1.0))
    h = jnp.concatenate([a0_ref[...], a1_ref[...]], axis=1) * nd + b_ref[...]
    h = jnp.maximum(h, 0.0)
    o_ref[...] = jnp.dot(h, w_ref[...], preferred_element_type=jnp.float32) * ns


def _tc3_body(a0_ref, a1_ref, b_ref, dd_ref, o_ref):
    nd = lax.rsqrt(jnp.maximum(dd_ref[...], 1.0))
    h = jnp.concatenate([a0_ref[...], a1_ref[...]], axis=1) * nd + b_ref[...]
    o_ref[...] = jnp.maximum(h, 0.0)


def _tc1(feat, w, deg_src):
    return pl.pallas_call(
        _tc1_body,
        grid=(NPAD // _BN, 2),
        in_specs=[
            pl.BlockSpec((_BN, D), lambda i, c: (i, 0)),
            pl.BlockSpec((D, DH), lambda i, c: (0, c)),
            pl.BlockSpec((_BN, 1), lambda i, c: (i, 0)),
        ],
        out_specs=pl.BlockSpec((_BN, DH), lambda i, c: (c * (NPAD // _BN) + i, 0)),
        out_shape=jax.ShapeDtypeStruct((2 * NPAD, DH), jnp.float32),
    )(feat, w, deg_src)


def _tc2(agg, w, b, deg_dst, deg_src):
    nb = NPAD // _BN
    return pl.pallas_call(
        _tc2_body,
        grid=(nb, 2),
        in_specs=[
            pl.BlockSpec((_BN, DH), lambda i, c: (i, 0)),
            pl.BlockSpec((_BN, DH), lambda i, c: (nb + i, 0)),
            pl.BlockSpec((D, DH), lambda i, c: (0, c)),
            pl.BlockSpec((1, D), lambda i, c: (0, 0)),
            pl.BlockSpec((_BN, 1), lambda i, c: (i, 0)),
            pl.BlockSpec((_BN, 1), lambda i, c: (i, 0)),
        ],
        out_specs=pl.BlockSpec((_BN, DH), lambda i, c: (c * nb + i, 0)),
        out_shape=jax.ShapeDtypeStruct((2 * NPAD, DH), jnp.float32),
    )(agg, agg, w, b, deg_dst, deg_src)


def _tc3(agg, b, deg_dst):
    nb = NPAD // _BN
    return pl.pallas_call(
        _tc3_body,
        grid=(nb,),
        in_specs=[
            pl.BlockSpec((_BN, DH), lambda i: (i, 0)),
            pl.BlockSpec((_BN, DH), lambda i: (nb + i, 0)),
            pl.BlockSpec((1, D), lambda i: (0, 0)),
            pl.BlockSpec((_BN, 1), lambda i: (i, 0)),
        ],
        out_specs=pl.BlockSpec((_BN, D), lambda i: (i, 0)),
        out_shape=jax.ShapeDtypeStruct((N, D), jnp.float32),
    )(agg, agg, b, deg_dst)


def kernel(feat, edge_index, W1, b1, W2, b2):
    src = edge_index[0]
    dst = edge_index[1]
    pad = EPAD - E
    src_p = jnp.concatenate([src, jnp.full((pad,), DUMMY, jnp.int32)])
    dst_p = jnp.concatenate([dst, jnp.full((pad,), DUMMY, jnp.int32)])
    src4 = src_p.reshape(16, NBLK, BI, CHUNK)
    dst4 = dst_p.reshape(16, NBLK, BI, CHUNK)
    sdidx = jnp.stack([
        jnp.stack([src4, dst4], axis=3),
        jnp.stack([src4 + NPAD, dst4], axis=3),
    ])

    degs = _degree_kernel(sdidx)
    deg_src = degs[0].reshape(NPAD, 1)
    deg_dst = degs[1].reshape(NPAD, 1)

    b1r = b1.reshape(1, D)
    b2r = b2.reshape(1, D)
    hs1 = _tc1(feat, W1, deg_src)
    agg1 = _agg_kernel(hs1, sdidx)
    hs2 = _tc2(agg1, W2, b1r, deg_dst, deg_src)
    agg2 = _agg_kernel(hs2, sdidx)
    return _tc3(agg2, b2r, deg_dst)


# speedup vs baseline: 1.9405x; 1.5573x over previous
import functools

import jax
import jax.numpy as jnp
import numpy as np
from jax import lax
from jax.experimental import pallas as pl
from jax.experimental.pallas import tpu as pltpu
from jax.experimental.pallas import tpu_sc as plsc

N = 10000
E = 160000
D = 256
DH = 128
NPAD = 10240
ROWS_PER_TILE = NPAD // 16
CHUNK = 84
BI = 10
NBLK = 12
CPB = 80
NBUF = 2
PIPE = NBUF - 1
NCH = BI * NBLK
E_TILE = NCH * CHUNK
EPAD = 16 * E_TILE
DUMMY = N

_MESH = plsc.VectorSubcoreMesh(core_axis_name="c", subcore_axis_name="s",
                               num_cores=2, num_subcores=16)


def _degree_body(sdidx_hbm, out_hbm, idx_v, ones_v, row_v, hist_s):
    c = lax.axis_index("c")
    s = lax.axis_index("s")

    def _fill_ones(i, carry):
        ones_v[pl.ds(i * 16, 16)] = jnp.ones((16,), jnp.float32)
        return carry

    lax.fori_loop(0, (CHUNK + 15) // 16, _fill_ones, 0)

    def _fill_zero(i, carry):
        row_v[pl.ds(i * 16, 16)] = jnp.zeros((16,), jnp.float32)
        return carry

    lax.fori_loop(0, ROWS_PER_TILE // 16, _fill_zero, 0)
    pltpu.sync_copy(row_v, hist_s.at[pl.ds(s * ROWS_PER_TILE, ROWS_PER_TILE)])
    pltpu.sync_copy(sdidx_hbm.at[0, s], idx_v)
    plsc.subcore_barrier()

    def _accum(b, carry):
        for i in range(BI):
            pltpu.sync_copy(ones_v.at[pl.ds(0, CHUNK)],
                            hist_s.at[idx_v.at[b, i, c]], add=True)
        return carry

    lax.fori_loop(0, NBLK, _accum, 0)
    plsc.subcore_barrier()
    pltpu.sync_copy(hist_s.at[pl.ds(s * ROWS_PER_TILE, ROWS_PER_TILE)], row_v)
    pltpu.sync_copy(row_v, out_hbm.at[c, pl.ds(s * ROWS_PER_TILE, ROWS_PER_TILE)])


_degree_kernel = functools.partial(
    pl.kernel,
    out_type=jax.ShapeDtypeStruct((2, NPAD), jnp.float32),
    mesh=_MESH,
    scratch_types=[
        pltpu.VMEM((NBLK, BI, 2, CHUNK), jnp.int32),
        pltpu.VMEM((16 * ((CHUNK + 15) // 16),), jnp.float32),
        pltpu.VMEM((ROWS_PER_TILE,), jnp.float32),
        pltpu.VMEM_SHARED((NPAD,), jnp.float32),
    ],
)(_degree_body)


def _agg_body(table_hbm, sdidx_hbm, out_hbm,
              sd_v0, sd_v1, buf_a, buf_b, agg_s,
              sem_g0, sem_g1, sem_i0, sem_i1):
    c = lax.axis_index("c")
    s = lax.axis_index("s")
    bufs = (buf_a, buf_b)
    gsems = (sem_g0, sem_g1)
    sds = (sd_v0, sd_v1)
    isems = (sem_i0, sem_i1)

    def _zero_row(i, carry):
        for l in range(DH // 16):
            buf_a[i, pl.ds(l * 16, 16)] = jnp.zeros((16,), jnp.float32)
        return carry

    lax.fori_loop(0, CPB, _zero_row, 0)
    for k in range(ROWS_PER_TILE // CPB):
        pltpu.sync_copy(buf_a.at[pl.ds(0, CPB)],
                        agg_s.at[pl.ds(s * ROWS_PER_TILE + k * CPB, CPB)])
    plsc.subcore_barrier()

    def _gather(sd, i, p):
        pltpu.async_copy(table_hbm.at[sd.at[i, 0]], bufs[p], gsems[p])

    def _wait_gather(sd, i, p):
        pltpu.make_async_copy(table_hbm.at[sd.at[i, 0]], bufs[p], gsems[p]).wait()

    pltpu.sync_copy(sdidx_hbm.at[c, s, 0], sd_v0)
    pltpu.async_copy(sdidx_hbm.at[c, s, 1], sd_v1, sem_i1)
    for i in range(PIPE):
        _gather(sd_v0, i, i)

    def _dblk(t, carry):
        for half in range(2):
            sd = sds[half]
            nxt = sds[(half + 1) % 2]
            blk = 2 * t + half
            last = half == 1
            for i in range(BI):
                p = i % NBUF
                if i < BI - PIPE:
                    _gather(sd, i + PIPE, (i + PIPE) % NBUF)
                else:
                    ii = i + PIPE - BI

                    def _issue_next(ii=ii):
                        if ii == 0:
                            pltpu.make_async_copy(
                                sdidx_hbm.at[c, s, blk + 1], nxt,
                                isems[(half + 1) % 2]).wait()
                        _gather(nxt, ii, ii % NBUF)

                    if not last:
                        _issue_next()
                    else:
                        pl.when(t < NBLK // 2 - 1)(_issue_next)
                _wait_gather(sd, i, p)
                pltpu.sync_copy(bufs[p], agg_s.at[sd.at[i, 1]], add=True)

            def _prefetch():
                pltpu.async_copy(sdidx_hbm.at[c, s, blk + 2], sd,
                                 isems[half])

            pl.when(blk + 2 < NBLK)(_prefetch)
        return carry

    lax.fori_loop(0, NBLK // 2, _dblk, 0)

    plsc.subcore_barrier()
    for k in range(ROWS_PER_TILE // CPB):
        off = s * ROWS_PER_TILE + k * CPB
        pltpu.sync_copy(agg_s.at[pl.ds(off, CPB)], buf_a.at[pl.ds(0, CPB)])
        pltpu.sync_copy(buf_a.at[pl.ds(0, CPB)],
                        out_hbm.at[pl.ds(c * NPAD + off, CPB)])


_agg_kernel = functools.partial(
    pl.kernel,
    out_type=jax.ShapeDtypeStruct((2 * NPAD, DH), jnp.float32),
    mesh=_MESH,
    scratch_types=[
        pltpu.VMEM((BI, 2, CHUNK), jnp.int32),
        pltpu.VMEM((BI, 2, CHUNK), jnp.int32),
        pltpu.VMEM((CHUNK, DH), jnp.float32),
        pltpu.VMEM((CHUNK, DH), jnp.float32),
        pltpu.VMEM_SHARED((NPAD, DH), jnp.float32),
        pltpu.SemaphoreType.DMA,
        pltpu.SemaphoreType.DMA,
        pltpu.SemaphoreType.DMA,
        pltpu.SemaphoreType.DMA,
    ],
)(_agg_body)


_BN = 640


def _tc1_body(f_ref, w_ref, d_ref, o_ref):
    norm = lax.rsqrt(jnp.maximum(d_ref[...], 1.0))
    o_ref[...] = jnp.dot(f_ref[...], w_ref[...],
                         preferred_element_type=jnp.float32) * norm


def _tc2_body(a0_ref, a1_ref, w_ref, b_ref, dd_ref, ds_ref, o_ref):
    nd = lax.rsqrt(jnp.maximum(dd_ref[...], 1.0))
    ns = lax.rsqrt(jnp.maximum(ds_ref[...], 1.0))
    h = jnp.concatenate([a0_ref[...], a1_ref[...]], axis=1) * nd + b_ref[...]
    h = jnp.maximum(h, 0.0)
    o_ref[...] = jnp.dot(h, w_ref[...], preferred_element_type=jnp.float32) * ns


def _tc3_body(a0_ref, a1_ref, b_ref, dd_ref, o_ref):
    nd = lax.rsqrt(jnp.maximum(dd_ref[...], 1.0))
    h = jnp.concatenate([a0_ref[...], a1_ref[...]], axis=1) * nd + b_ref[...]
    o_ref[...] = jnp.maximum(h, 0.0)


def _tc1(feat, w, deg_src):
    return pl.pallas_call(
        _tc1_body,
        grid=(NPAD // _BN, 2),
        in_specs=[
            pl.BlockSpec((_BN, D), lambda i, c: (i, 0)),
            pl.BlockSpec((D, DH), lambda i, c: (0, c)),
            pl.BlockSpec((_BN, 1), lambda i, c: (i, 0)),
        ],
        out_specs=pl.BlockSpec((_BN, DH), lambda i, c: (c * (NPAD // _BN) + i, 0)),
        out_shape=jax.ShapeDtypeStruct((2 * NPAD, DH), jnp.float32),
    )(feat, w, deg_src)


def _tc2(agg, w, b, deg_dst, deg_src):
    nb = NPAD // _BN
    return pl.pallas_call(
        _tc2_body,
        grid=(nb, 2),
        in_specs=[
            pl.BlockSpec((_BN, DH), lambda i, c: (i, 0)),
            pl.BlockSpec((_BN, DH), lambda i, c: (nb + i, 0)),
            pl.BlockSpec((D, DH), lambda i, c: (0, c)),
            pl.BlockSpec((1, D), lambda i, c: (0, 0)),
            pl.BlockSpec((_BN, 1), lambda i, c: (i, 0)),
            pl.BlockSpec((_BN, 1), lambda i, c: (i, 0)),
        ],
        out_specs=pl.BlockSpec((_BN, DH), lambda i, c: (c * nb + i, 0)),
        out_shape=jax.ShapeDtypeStruct((2 * NPAD, DH), jnp.float32),
    )(agg, agg, w, b, deg_dst, deg_src)


def _tc3(agg, b, deg_dst):
    nb = NPAD // _BN
    return pl.pallas_call(
        _tc3_body,
        grid=(nb,),
        in_specs=[
            pl.BlockSpec((_BN, DH), lambda i: (i, 0)),
            pl.BlockSpec((_BN, DH), lambda i: (nb + i, 0)),
            pl.BlockSpec((1, D), lambda i: (0, 0)),
            pl.BlockSpec((_BN, 1), lambda i: (i, 0)),
        ],
        out_specs=pl.BlockSpec((_BN, D), lambda i: (i, 0)),
        out_shape=jax.ShapeDtypeStruct((N, D), jnp.float32),
    )(agg, agg, b, deg_dst)


def kernel(feat, edge_index, W1, b1, W2, b2):
    src = edge_index[0]
    dst = edge_index[1]
    pad = EPAD - E
    src_p = jnp.concatenate([src, jnp.full((pad,), DUMMY, jnp.int32)])
    dst_p = jnp.concatenate([dst, jnp.full((pad,), DUMMY, jnp.int32)])
    src4 = src_p.reshape(16, NBLK, BI, CHUNK)
    dst4 = dst_p.reshape(16, NBLK, BI, CHUNK)
    sdidx = jnp.stack([
        jnp.stack([src4, dst4], axis=3),
        jnp.stack([src4 + NPAD, dst4], axis=3),
    ])

    degs = _degree_kernel(sdidx)
    deg_src = degs[0].reshape(NPAD, 1)
    deg_dst = degs[1].reshape(NPAD, 1)

    b1r = b1.reshape(1, D)
    b2r = b2.reshape(1, D)
    hs1 = _tc1(feat, W1, deg_src)
    agg1 = _agg_kernel(hs1, sdidx)
    hs2 = _tc2(agg1, W2, b1r, deg_dst, deg_src)
    agg2 = _agg_kernel(hs2, sdidx)
    return _tc3(agg2, b2r, deg_dst)
